# Initial kernel scaffold; baseline (speedup 1.0000x reference)
import jax
import jax.numpy as jnp
from jax.experimental import pallas as pl


def kernel(img, grid):
    raise NotImplementedError("write your pallas kernel here")


# speedup vs baseline: 1.7052x; 1.7052x over previous
import functools

import jax
import jax.numpy as jnp
from jax import lax
from jax.experimental import pallas as pl
from jax.experimental.pallas import tpu as pltpu
from jax.experimental.pallas import tpu_sc as plsc

N, C, H, W = 4, 96, 384, 384
HW = H * W
NW = 32
GP = 128
NG = (HW // GP) // NW
TB = 2048


def _transpose_body(x_ref, o_ref):
    o_ref[0] = x_ref[0].T


def _make_table(img):
    imgr = img.reshape(N, C, HW)
    return pl.pallas_call(
        _transpose_body,
        grid=(N, HW // TB),
        in_specs=[pl.BlockSpec((1, C, TB), lambda n, i: (n, 0, i))],
        out_specs=pl.BlockSpec((1, TB, C), lambda n, i: (n, i, 0)),
        out_shape=jax.ShapeDtypeStruct((N, HW, C), jnp.float32),
    )(imgr)


def _sc_body(table, grid_f, out, grid_v, idx_v, w_v, rows_v, out_v, sem):
    wid = lax.axis_index("s") * 2 + lax.axis_index("c")
    iota = lax.iota(jnp.int32, 16)

    for n in range(N):
        goff = n * (HW * 2) + wid * (NG * GP * 2)
        pltpu.sync_copy(grid_f.at[pl.ds(goff, NG * GP * 2)], grid_v)

        def body(g, carry, n=n):
            p_sample = (wid * NG + g) * GP

            for sv in range(8):
                base = g * (GP * 2) + sv * 32
                gxr = plsc.load_gather(grid_v, [base + 2 * iota])
                gyr = plsc.load_gather(grid_v, [base + 2 * iota + 1])
                gx = (gxr + 1.0) * (0.5 * (W - 1))
                gy = (gyr + 1.0) * (0.5 * (H - 1))
                x0 = jnp.minimum(gx.astype(jnp.int32), W - 2)
                y0 = jnp.minimum(gy.astype(jnp.int32), H - 2)
                wx1 = gx - x0.astype(jnp.float32)
                wy1 = gy - y0.astype(jnp.float32)
                wx0 = 1.0 - wx1
                wy0 = 1.0 - wy1
                r00 = y0 * W + x0 + (n * HW)
                sl = pl.ds(sv * 16, 16)
                idx_v[0, sl] = r00
                idx_v[1, sl] = r00 + 1
                idx_v[2, sl] = r00 + W
                idx_v[3, sl] = r00 + (W + 1)
                w_v[0, sl] = wx0 * wy0
                w_v[1, sl] = wx1 * wy0
                w_v[2, sl] = wx0 * wy1
                w_v[3, sl] = wx1 * wy1

            cps = [
                pltpu.async_copy(
                    table.at[idx_v.at[k]], rows_v.at[pl.ds(k * GP, GP)], sem
                )
                for k in range(4)
            ]
            for cp in cps:
                cp.wait()

            for sv in range(8):
                slp = pl.ds(sv * 16, 16)
                w00 = w_v[0, slp]
                w01 = w_v[1, slp]
                w10 = w_v[2, slp]
                w11 = w_v[3, slp]
                pid = iota + (sv * 16)
                r0 = pid
                r1 = pid + GP
                r2 = pid + 2 * GP
                r3 = pid + 3 * GP

                def cbody(cb, carry2, w00=w00, w01=w01, w10=w10, w11=w11,
                          r0=r0, r1=r1, r2=r2, r3=r3, slp=slp):
                    for j in range(4):
                        c = cb * 4 + j
                        chv = jnp.zeros((16,), jnp.int32) + c
                        a00 = plsc.load_gather(rows_v, [r0, chv])
                        a01 = plsc.load_gather(rows_v, [r1, chv])
                        a10 = plsc.load_gather(rows_v, [r2, chv])
                        a11 = plsc.load_gather(rows_v, [r3, chv])
                        out_v[c, slp] = a00 * w00 + a01 * w01 + a10 * w10 + a11 * w11
                    return carry2

                lax.fori_loop(0, C // 4, cbody, 0)

            pltpu.sync_copy(out_v, out.at[n, :, pl.ds(p_sample, GP)])
            return carry

        lax.fori_loop(0, NG, body, 0)


@functools.partial(
    pl.kernel,
    out_type=jax.ShapeDtypeStruct((N, C, HW), jnp.float32),
    mesh=plsc.VectorSubcoreMesh(core_axis_name="c", subcore_axis_name="s"),
    compiler_params=pltpu.CompilerParams(
        needs_layout_passes=False, use_tc_tiling_on_sc=False
    ),
    scratch_types=[
        pltpu.VMEM((NG * GP * 2,), jnp.float32),
        pltpu.VMEM((4, GP), jnp.int32),
        pltpu.VMEM((4, GP), jnp.float32),
        pltpu.VMEM((4 * GP, C), jnp.float32),
        pltpu.VMEM((C, GP), jnp.float32),
        pltpu.SemaphoreType.DMA,
    ],
)
def _sc_sample(table, grid_f, out, grid_v, idx_v, w_v, rows_v, out_v, sem):
    _sc_body(table, grid_f, out, grid_v, idx_v, w_v, rows_v, out_v, sem)


def kernel(img, grid):
    table = _make_table(img).reshape(N * HW, C)
    grid_f = grid.reshape(N * HW * 2)
    out = _sc_sample(table, grid_f)
    return out.reshape(N, C, H, W)


# speedup vs baseline: 2.1173x; 1.2417x over previous
import functools

import jax
import jax.numpy as jnp
from jax import lax
from jax.experimental import pallas as pl
from jax.experimental.pallas import tpu as pltpu
from jax.experimental.pallas import tpu_sc as plsc

N, C, H, W = 4, 96, 384, 384
HW = H * W
NW = 32
GP = 128
NG = (HW // GP) // NW
TB = 2048


def _transpose_body(x_ref, o_ref):
    o_ref[0] = x_ref[0].T


def _make_table(img):
    imgr = img.reshape(N, C, HW)
    return pl.pallas_call(
        _transpose_body,
        grid=(N, HW // TB),
        in_specs=[pl.BlockSpec((1, C, TB), lambda n, i: (n, 0, i))],
        out_specs=pl.BlockSpec((1, TB, C), lambda n, i: (n, i, 0)),
        out_shape=jax.ShapeDtypeStruct((N, HW, C), jnp.float32),
    )(imgr)


def _sc_body(table, grid_f, out, grid_v, idx_v, w_v, rows_v, out_v, sem):
    wid = lax.axis_index("s") * 2 + lax.axis_index("c")
    iota = lax.iota(jnp.int32, 16)

    for n in range(N):
        goff = n * (HW * 2) + wid * (NG * GP * 2)
        pltpu.sync_copy(grid_f.at[pl.ds(goff, NG * GP * 2)], grid_v)

        def body(g, carry, n=n):
            p_sample = (wid * NG + g) * GP

            for sv in range(8):
                base = g * (GP * 2) + sv * 32
                gxr = plsc.load_gather(grid_v, [base + 2 * iota])
                gyr = plsc.load_gather(grid_v, [base + 2 * iota + 1])
                gx = (gxr + 1.0) * (0.5 * (W - 1))
                gy = (gyr + 1.0) * (0.5 * (H - 1))
                x0 = jnp.minimum(gx.astype(jnp.int32), W - 2)
                y0 = jnp.minimum(gy.astype(jnp.int32), H - 2)
                wx1 = gx - x0.astype(jnp.float32)
                wy1 = gy - y0.astype(jnp.float32)
                wx0 = 1.0 - wx1
                wy0 = 1.0 - wy1
                r00 = y0 * W + x0 + (n * HW)
                sl = pl.ds(sv * 16, 16)
                idx_v[0, sl] = r00
                idx_v[1, sl] = r00 + 1
                idx_v[2, sl] = r00 + W
                idx_v[3, sl] = r00 + (W + 1)
                w_v[0, sl] = wx0 * wy0
                w_v[1, sl] = wx1 * wy0
                w_v[2, sl] = wx0 * wy1
                w_v[3, sl] = wx1 * wy1

            cps = [
                pltpu.async_copy(
                    table.at[idx_v.at[k]], rows_v.at[pl.ds(k * GP, GP)], sem
                )
                for k in range(4)
            ]
            for cp in cps:
                cp.wait()

            for sv in range(8):
                slp = pl.ds(sv * 16, 16)
                w00 = w_v[0, slp]
                w01 = w_v[1, slp]
                w10 = w_v[2, slp]
                w11 = w_v[3, slp]
                pid = iota + (sv * 16)
                r0 = pid
                r1 = pid + GP
                r2 = pid + 2 * GP
                r3 = pid + 3 * GP

                @plsc.parallel_loop(0, C, step=4, unroll=2)
                def cbody(cb, w00=w00, w01=w01, w10=w10, w11=w11,
                          r0=r0, r1=r1, r2=r2, r3=r3, slp=slp):
                    czero = jnp.zeros((16,), jnp.int32)
                    for j in range(4):
                        c = cb + j
                        chv = czero + c
                        a00 = plsc.load_gather(rows_v, [r0, chv])
                        a01 = plsc.load_gather(rows_v, [r1, chv])
                        a10 = plsc.load_gather(rows_v, [r2, chv])
                        a11 = plsc.load_gather(rows_v, [r3, chv])
                        out_v[c, slp] = a00 * w00 + a01 * w01 + a10 * w10 + a11 * w11

            pltpu.sync_copy(out_v, out.at[n, :, pl.ds(p_sample, GP)])
            return carry

        lax.fori_loop(0, NG, body, 0)


@functools.partial(
    pl.kernel,
    out_type=jax.ShapeDtypeStruct((N, C, HW), jnp.float32),
    mesh=plsc.VectorSubcoreMesh(core_axis_name="c", subcore_axis_name="s"),
    compiler_params=pltpu.CompilerParams(
        needs_layout_passes=False, use_tc_tiling_on_sc=False
    ),
    scratch_types=[
        pltpu.VMEM((NG * GP * 2,), jnp.float32),
        pltpu.VMEM((4, GP), jnp.int32),
        pltpu.VMEM((4, GP), jnp.float32),
        pltpu.VMEM((4 * GP, C), jnp.float32),
        pltpu.VMEM((C, GP), jnp.float32),
        pltpu.SemaphoreType.DMA,
    ],
)
def _sc_sample(table, grid_f, out, grid_v, idx_v, w_v, rows_v, out_v, sem):
    _sc_body(table, grid_f, out, grid_v, idx_v, w_v, rows_v, out_v, sem)


def kernel(img, grid):
    table = _make_table(img).reshape(N * HW, C)
    grid_f = grid.reshape(N * HW * 2)
    out = _sc_sample(table, grid_f)
    return out.reshape(N, C, H, W)


# speedup vs baseline: 4.0794x; 1.9267x over previous
import functools

import jax
import jax.numpy as jnp
from jax import lax
from jax.experimental import pallas as pl
from jax.experimental.pallas import tpu as pltpu
from jax.experimental.pallas import tpu_sc as plsc

N, C, H, W = 4, 96, 384, 384
HW = H * W
NW = 32
GP = 128
NG = (HW // GP) // NW
TB = 2048


def _fwd_transpose_body(x_ref, o_ref):
    o_ref[0] = x_ref[0].T


def _make_table(img):
    imgr = img.reshape(N, C, HW)
    return pl.pallas_call(
        _fwd_transpose_body,
        grid=(N, HW // TB),
        in_specs=[pl.BlockSpec((1, C, TB), lambda n, i: (n, 0, i))],
        out_specs=pl.BlockSpec((1, TB, C), lambda n, i: (n, i, 0)),
        out_shape=jax.ShapeDtypeStruct((N, HW, C), jnp.float32),
    )(imgr)


def _out_transpose(outp):
    return pl.pallas_call(
        _fwd_transpose_body,
        grid=(N, HW // TB),
        in_specs=[pl.BlockSpec((1, TB, C), lambda n, i: (n, i, 0))],
        out_specs=pl.BlockSpec((1, C, TB), lambda n, i: (n, 0, i)),
        out_shape=jax.ShapeDtypeStruct((N, C, HW), jnp.float32),
    )(outp)


def _sc_body(table, grid_f, out, grid_v, idx_v, w_v, rows_v, out_v, sem):
    wid = lax.axis_index("s") * 2 + lax.axis_index("c")
    iota = lax.iota(jnp.int32, 16)

    for n in range(N):
        goff = n * (HW * 2) + wid * (NG * GP * 2)
        pltpu.sync_copy(grid_f.at[pl.ds(goff, NG * GP * 2)], grid_v)

        def body(g, carry, n=n):
            p_sample = (wid * NG + g) * GP

            for sv in range(8):
                base = g * (GP * 2) + sv * 32
                gxr = plsc.load_gather(grid_v, [base + 2 * iota])
                gyr = plsc.load_gather(grid_v, [base + 2 * iota + 1])
                gx = (gxr + 1.0) * (0.5 * (W - 1))
                gy = (gyr + 1.0) * (0.5 * (H - 1))
                x0 = jnp.minimum(gx.astype(jnp.int32), W - 2)
                y0 = jnp.minimum(gy.astype(jnp.int32), H - 2)
                wx1 = gx - x0.astype(jnp.float32)
                wy1 = gy - y0.astype(jnp.float32)
                wx0 = 1.0 - wx1
                wy0 = 1.0 - wy1
                r00 = y0 * W + x0 + (n * HW)
                sl = pl.ds(sv * 16, 16)
                idx_v[0, sl] = r00
                idx_v[1, sl] = r00 + 1
                idx_v[2, sl] = r00 + W
                idx_v[3, sl] = r00 + (W + 1)
                pid4 = (iota + sv * 16) * 4
                plsc.store_scatter(w_v, [pid4], wx0 * wy0)
                plsc.store_scatter(w_v, [pid4 + 1], wx1 * wy0)
                plsc.store_scatter(w_v, [pid4 + 2], wx0 * wy1)
                plsc.store_scatter(w_v, [pid4 + 3], wx1 * wy1)

            cps = [
                pltpu.async_copy(
                    table.at[idx_v.at[k]], rows_v.at[pl.ds(k * GP, GP)], sem
                )
                for k in range(4)
            ]
            for cp in cps:
                cp.wait()

            @plsc.parallel_loop(0, GP, step=4, unroll=2)
            def pbody(p):
                wv = w_v[pl.ds(p * 4, 16)]
                for q in range(4):
                    pq = p + q
                    w00 = wv[4 * q]
                    w01 = wv[4 * q + 1]
                    w10 = wv[4 * q + 2]
                    w11 = wv[4 * q + 3]
                    for j in range(C // 16):
                        slc = pl.ds(j * 16, 16)
                        a00 = rows_v[pq, slc]
                        a01 = rows_v[pq + GP, slc]
                        a10 = rows_v[pq + 2 * GP, slc]
                        a11 = rows_v[pq + 3 * GP, slc]
                        out_v[pq, slc] = a00 * w00 + a01 * w01 + a10 * w10 + a11 * w11

            pltpu.sync_copy(out_v, out.at[pl.ds(n * HW + p_sample, GP)])
            return carry

        lax.fori_loop(0, NG, body, 0)


@functools.partial(
    pl.kernel,
    out_type=jax.ShapeDtypeStruct((N * HW, C), jnp.float32),
    mesh=plsc.VectorSubcoreMesh(core_axis_name="c", subcore_axis_name="s"),
    compiler_params=pltpu.CompilerParams(
        needs_layout_passes=False, use_tc_tiling_on_sc=False
    ),
    scratch_types=[
        pltpu.VMEM((NG * GP * 2,), jnp.float32),
        pltpu.VMEM((4, GP), jnp.int32),
        pltpu.VMEM((4 * GP,), jnp.float32),
        pltpu.VMEM((4 * GP, C), jnp.float32),
        pltpu.VMEM((GP, C), jnp.float32),
        pltpu.SemaphoreType.DMA,
    ],
)
def _sc_sample(table, grid_f, out, grid_v, idx_v, w_v, rows_v, out_v, sem):
    _sc_body(table, grid_f, out, grid_v, idx_v, w_v, rows_v, out_v, sem)


def kernel(img, grid):
    table = _make_table(img).reshape(N * HW, C)
    grid_f = grid.reshape(N * HW * 2)
    outp = _sc_sample(table, grid_f)
    out = _out_transpose(outp.reshape(N, HW, C))
    return out.reshape(N, C, H, W)


# speedup vs baseline: 6.3868x; 1.5656x over previous
import functools

import jax
import jax.numpy as jnp
from jax import lax
from jax.experimental import pallas as pl
from jax.experimental.pallas import tpu as pltpu
from jax.experimental.pallas import tpu_sc as plsc

N, C, H, W = 4, 96, 384, 384
CP = 128
HW = H * W
NW = 32
GP = 128
NG = (HW // GP) // NW
HB = 8
TB = HB * W


def _fwd_body(x_ref, o_ref):
    for r in range(HB):
        o_ref[pl.ds(r * W, W), 0:C] = x_ref[0, :, r, :].T


def _make_table(img):
    return pl.pallas_call(
        _fwd_body,
        grid=(N, H // HB),
        in_specs=[pl.BlockSpec((1, C, HB, W), lambda n, i: (n, 0, i, 0))],
        out_specs=pl.BlockSpec((TB, CP), lambda n, i: (n * (H // HB) + i, 0)),
        out_shape=jax.ShapeDtypeStruct((N * HW, CP), jnp.float32),
    )(img)


def _bwd_body(x_ref, o_ref):
    for r in range(HB):
        o_ref[0, :, r, :] = x_ref[pl.ds(r * W, W), 0:C].T


def _out_transpose(outp):
    return pl.pallas_call(
        _bwd_body,
        grid=(N, H // HB),
        in_specs=[pl.BlockSpec((TB, CP), lambda n, i: (n * (H // HB) + i, 0))],
        out_specs=pl.BlockSpec((1, C, HB, W), lambda n, i: (n, 0, i, 0)),
        out_shape=jax.ShapeDtypeStruct((N, C, H, W), jnp.float32),
    )(outp)


def _sc_body(table, grid_f, out, grid_v, idx_v, w_v, rows_v, out_v, sem):
    wid = lax.axis_index("s") * 2 + lax.axis_index("c")
    iota = lax.iota(jnp.int32, 16)

    for n in range(N):
        goff = n * (HW * 2) + wid * (NG * GP * 2)
        pltpu.sync_copy(grid_f.at[pl.ds(goff, NG * GP * 2)], grid_v)

        def body(g, carry, n=n):
            p_sample = (wid * NG + g) * GP

            for sv in range(8):
                base = g * (GP * 2) + sv * 32
                gxr = plsc.load_gather(grid_v, [base + 2 * iota])
                gyr = plsc.load_gather(grid_v, [base + 2 * iota + 1])
                gx = (gxr + 1.0) * (0.5 * (W - 1))
                gy = (gyr + 1.0) * (0.5 * (H - 1))
                x0 = jnp.minimum(gx.astype(jnp.int32), W - 2)
                y0 = jnp.minimum(gy.astype(jnp.int32), H - 2)
                wx1 = gx - x0.astype(jnp.float32)
                wy1 = gy - y0.astype(jnp.float32)
                wx0 = 1.0 - wx1
                wy0 = 1.0 - wy1
                r00 = y0 * W + x0 + (n * HW)
                sl = pl.ds(sv * 16, 16)
                idx_v[0, sl] = r00
                idx_v[1, sl] = r00 + 1
                idx_v[2, sl] = r00 + W
                idx_v[3, sl] = r00 + (W + 1)
                pid4 = (iota + sv * 16) * 4
                plsc.store_scatter(w_v, [pid4], wx0 * wy0)
                plsc.store_scatter(w_v, [pid4 + 1], wx1 * wy0)
                plsc.store_scatter(w_v, [pid4 + 2], wx0 * wy1)
                plsc.store_scatter(w_v, [pid4 + 3], wx1 * wy1)

            cps = [
                pltpu.async_copy(
                    table.at[idx_v.at[k]], rows_v.at[pl.ds(k * GP, GP)], sem
                )
                for k in range(4)
            ]
            for cp in cps:
                cp.wait()

            @plsc.parallel_loop(0, GP, step=4, unroll=2)
            def pbody(p):
                wv = w_v[pl.ds(p * 4, 16)]
                for q in range(4):
                    pq = p + q
                    w00 = wv[4 * q]
                    w01 = wv[4 * q + 1]
                    w10 = wv[4 * q + 2]
                    w11 = wv[4 * q + 3]
                    for j in range(C // 16):
                        slc = pl.ds(j * 16, 16)
                        a00 = rows_v[pq, slc]
                        a01 = rows_v[pq + GP, slc]
                        a10 = rows_v[pq + 2 * GP, slc]
                        a11 = rows_v[pq + 3 * GP, slc]
                        out_v[pq, slc] = a00 * w00 + a01 * w01 + a10 * w10 + a11 * w11

            pltpu.sync_copy(out_v, out.at[pl.ds(n * HW + p_sample, GP)])
            return carry

        lax.fori_loop(0, NG, body, 0)


@functools.partial(
    pl.kernel,
    out_type=jax.ShapeDtypeStruct((N * HW, CP), jnp.float32),
    mesh=plsc.VectorSubcoreMesh(core_axis_name="c", subcore_axis_name="s"),
    compiler_params=pltpu.CompilerParams(
        needs_layout_passes=False, use_tc_tiling_on_sc=True
    ),
    scratch_types=[
        pltpu.VMEM((NG * GP * 2,), jnp.float32),
        pltpu.VMEM((4, GP), jnp.int32),
        pltpu.VMEM((4 * GP,), jnp.float32),
        pltpu.VMEM((4 * GP, CP), jnp.float32),
        pltpu.VMEM((GP, CP), jnp.float32),
        pltpu.SemaphoreType.DMA,
    ],
)
def _sc_sample(table, grid_f, out, grid_v, idx_v, w_v, rows_v, out_v, sem):
    _sc_body(table, grid_f, out, grid_v, idx_v, w_v, rows_v, out_v, sem)


def kernel(img, grid):
    table = _make_table(img)
    grid_f = grid.reshape(N * HW * 2)
    outp = _sc_sample(table, grid_f)
    return _out_transpose(outp)


# speedup vs baseline: 7.8765x; 1.2332x over previous
import functools

import jax
import jax.numpy as jnp
from jax import lax
from jax.experimental import pallas as pl
from jax.experimental.pallas import tpu as pltpu
from jax.experimental.pallas import tpu_sc as plsc

N, C, H, W = 4, 96, 384, 384
CP = 128
HW = H * W
NW = 32
GP = 64
NG = (HW // GP) // NW
HB = 8
TB = HB * W


def _fwd_body(x_ref, o_ref):
    for r in range(HB):
        o_ref[pl.ds(r * W, W), 0:C] = x_ref[0, :, r, :].T


def _make_table(img):
    return pl.pallas_call(
        _fwd_body,
        grid=(N, H // HB),
        in_specs=[pl.BlockSpec((1, C, HB, W), lambda n, i: (n, 0, i, 0))],
        out_specs=pl.BlockSpec((TB, CP), lambda n, i: (n * (H // HB) + i, 0)),
        out_shape=jax.ShapeDtypeStruct((N * HW, CP), jnp.float32),
    )(img)


def _bwd_body(x_ref, o_ref):
    for r in range(HB):
        o_ref[0, :, r, :] = x_ref[pl.ds(r * W, W), 0:C].T


def _out_transpose(outp):
    return pl.pallas_call(
        _bwd_body,
        grid=(N, H // HB),
        in_specs=[pl.BlockSpec((TB, CP), lambda n, i: (n * (H // HB) + i, 0))],
        out_specs=pl.BlockSpec((1, C, HB, W), lambda n, i: (n, 0, i, 0)),
        out_shape=jax.ShapeDtypeStruct((N, C, H, W), jnp.float32),
    )(outp)


def _sc_body(table, grid_f, out, grid_v, idx_v0, idx_v1, w_v0, w_v1,
             rows_v0, rows_v1, out_v0, out_v1, sem0, sem1):
    wid = lax.axis_index("s") * 2 + lax.axis_index("c")
    iota = lax.iota(jnp.int32, 16)
    bufs = ((idx_v0, w_v0, rows_v0, out_v0, sem0),
            (idx_v1, w_v1, rows_v1, out_v1, sem1))

    def nbody(n, ncarry):
        goff = n * (HW * 2) + wid * (NG * GP * 2)
        pltpu.sync_copy(grid_f.at[pl.ds(goff, NG * GP * 2)], grid_v)

        def idx_fire(g, b, n=n):
            idx_v, w_v, rows_v, out_v, sem = bufs[b]
            for sv in range(GP // 16):
                base = g * (GP * 2) + sv * 32
                gxr = plsc.load_gather(grid_v, [base + 2 * iota])
                gyr = plsc.load_gather(grid_v, [base + 2 * iota + 1])
                gx = (gxr + 1.0) * (0.5 * (W - 1))
                gy = (gyr + 1.0) * (0.5 * (H - 1))
                x0 = jnp.minimum(gx.astype(jnp.int32), W - 2)
                y0 = jnp.minimum(gy.astype(jnp.int32), H - 2)
                wx1 = gx - x0.astype(jnp.float32)
                wy1 = gy - y0.astype(jnp.float32)
                wx0 = 1.0 - wx1
                wy0 = 1.0 - wy1
                r00 = y0 * W + x0 + (n * HW)
                sl = pl.ds(sv * 16, 16)
                idx_v[0, sl] = r00
                idx_v[1, sl] = r00 + 1
                idx_v[2, sl] = r00 + W
                idx_v[3, sl] = r00 + (W + 1)
                pid4 = (iota + sv * 16) * 4
                plsc.store_scatter(w_v, [pid4], wx0 * wy0)
                plsc.store_scatter(w_v, [pid4 + 1], wx1 * wy0)
                plsc.store_scatter(w_v, [pid4 + 2], wx0 * wy1)
                plsc.store_scatter(w_v, [pid4 + 3], wx1 * wy1)
            for k in range(4):
                pltpu.async_copy(
                    table.at[idx_v.at[k]],
                    rows_v.at[pl.ds(k * GP, GP)],
                    sem,
                )

        def drain_interp(g, b, n=n):
            idx_v, w_v, rows_v, out_v, sem = bufs[b]
            for k in range(4):
                pltpu.make_async_copy(
                    table.at[idx_v.at[k]],
                    rows_v.at[pl.ds(k * GP, GP)],
                    sem,
                ).wait()

            @plsc.parallel_loop(0, GP, step=4, unroll=2)
            def pbody(p):
                wv = w_v[pl.ds(p * 4, 16)]
                for q in range(4):
                    pq = p + q
                    w00 = wv[4 * q]
                    w01 = wv[4 * q + 1]
                    w10 = wv[4 * q + 2]
                    w11 = wv[4 * q + 3]
                    for j in range(C // 16):
                        slc = pl.ds(j * 16, 16)
                        a00 = rows_v[pq, slc]
                        a01 = rows_v[pq + GP, slc]
                        a10 = rows_v[pq + 2 * GP, slc]
                        a11 = rows_v[pq + 3 * GP, slc]
                        out_v[pq, slc] = (
                            a00 * w00 + a01 * w01 + a10 * w10 + a11 * w11
                        )

            p_sample = (wid * NG + g) * GP
            pltpu.sync_copy(out_v, out.at[pl.ds(n * HW + p_sample, GP)])

        idx_fire(0, 0)

        def body(g2, carry):
            gg = 2 * g2
            idx_fire(gg + 1, 1)
            drain_interp(gg, 0)
            idx_fire(gg + 2, 0)
            drain_interp(gg + 1, 1)
            return carry

        lax.fori_loop(0, NG // 2 - 1, body, 0)
        gg = NG - 2
        idx_fire(gg + 1, 1)
        drain_interp(gg, 0)
        drain_interp(gg + 1, 1)
        return ncarry

    lax.fori_loop(0, N, nbody, 0)


@functools.partial(
    pl.kernel,
    out_type=jax.ShapeDtypeStruct((N * HW, CP), jnp.float32),
    mesh=plsc.VectorSubcoreMesh(core_axis_name="c", subcore_axis_name="s"),
    compiler_params=pltpu.CompilerParams(
        needs_layout_passes=False, use_tc_tiling_on_sc=True
    ),
    scratch_types=[
        pltpu.VMEM((NG * GP * 2,), jnp.float32),
        pltpu.VMEM((4, GP), jnp.int32),
        pltpu.VMEM((4, GP), jnp.int32),
        pltpu.VMEM((4 * GP,), jnp.float32),
        pltpu.VMEM((4 * GP,), jnp.float32),
        pltpu.VMEM((4 * GP, CP), jnp.float32),
        pltpu.VMEM((4 * GP, CP), jnp.float32),
        pltpu.VMEM((GP, CP), jnp.float32),
        pltpu.VMEM((GP, CP), jnp.float32),
        pltpu.SemaphoreType.DMA,
        pltpu.SemaphoreType.DMA,
    ],
)
def _sc_sample(table, grid_f, out, grid_v, idx_v0, idx_v1, w_v0, w_v1,
               rows_v0, rows_v1, out_v0, out_v1, sem0, sem1):
    _sc_body(table, grid_f, out, grid_v, idx_v0, idx_v1, w_v0, w_v1,
             rows_v0, rows_v1, out_v0, out_v1, sem0, sem1)


def kernel(img, grid):
    table = _make_table(img)
    grid_f = grid.reshape(N * HW * 2)
    outp = _sc_sample(table, grid_f)
    return _out_transpose(outp)


# speedup vs baseline: 7.8833x; 1.0009x over previous
import functools

import jax
import jax.numpy as jnp
from jax import lax
from jax.experimental import pallas as pl
from jax.experimental.pallas import tpu as pltpu
from jax.experimental.pallas import tpu_sc as plsc

N, C, H, W = 4, 96, 384, 384
CP = 128
HW = H * W
NW = 32
GP = 64
NG = (HW // GP) // NW
HB = 8
TB = HB * W


def _fwd_body(x_ref, o_ref):
    for r in range(HB):
        o_ref[pl.ds(r * W, W), 0:C] = x_ref[0, :, r, :].T


def _make_table(img):
    return pl.pallas_call(
        _fwd_body,
        grid=(N, H // HB),
        in_specs=[pl.BlockSpec((1, C, HB, W), lambda n, i: (n, 0, i, 0))],
        out_specs=pl.BlockSpec((TB, CP), lambda n, i: (n * (H // HB) + i, 0)),
        out_shape=jax.ShapeDtypeStruct((N * HW, CP), jnp.float32),
    )(img)


def _bwd_body(x_ref, o_ref):
    for r in range(HB):
        o_ref[0, :, r, :] = x_ref[pl.ds(r * W, W), 0:C].T


def _out_transpose(outp):
    return pl.pallas_call(
        _bwd_body,
        grid=(N, H // HB),
        in_specs=[pl.BlockSpec((TB, CP), lambda n, i: (n * (H // HB) + i, 0))],
        out_specs=pl.BlockSpec((1, C, HB, W), lambda n, i: (n, 0, i, 0)),
        out_shape=jax.ShapeDtypeStruct((N, C, H, W), jnp.float32),
    )(outp)


def _sc_body(table, grid_f, out, grid_v, idx_v0, idx_v1, w_v0, w_v1,
             rows_v0, rows_v1, out_v0, out_v1, sem0, sem1):
    wid = lax.axis_index("s") * 2 + lax.axis_index("c")
    iota = lax.iota(jnp.int32, 16)
    bufs = ((idx_v0, w_v0, rows_v0, out_v0, sem0),
            (idx_v1, w_v1, rows_v1, out_v1, sem1))

    def nbody(n, ncarry):
        grow = n * (HW * 2 // 128) + wid * NG
        pltpu.sync_copy(grid_f.at[pl.ds(grow, NG)], grid_v)

        def idx_fire(g, b, n=n):
            idx_v, w_v, rows_v, out_v, sem = bufs[b]
            rowv = jnp.zeros((16,), jnp.int32) + g
            for sv in range(GP // 16):
                colv = sv * 32 + 2 * iota
                gxr = plsc.load_gather(grid_v, [rowv, colv])
                gyr = plsc.load_gather(grid_v, [rowv, colv + 1])
                gx = (gxr + 1.0) * (0.5 * (W - 1))
                gy = (gyr + 1.0) * (0.5 * (H - 1))
                x0 = jnp.minimum(gx.astype(jnp.int32), W - 2)
                y0 = jnp.minimum(gy.astype(jnp.int32), H - 2)
                wx1 = gx - x0.astype(jnp.float32)
                wy1 = gy - y0.astype(jnp.float32)
                wx0 = 1.0 - wx1
                wy0 = 1.0 - wy1
                r00 = y0 * W + x0 + (n * HW)
                sl = pl.ds(sv * 16, 16)
                idx_v[0, sl] = r00
                idx_v[1, sl] = r00 + 1
                idx_v[2, sl] = r00 + W
                idx_v[3, sl] = r00 + (W + 1)
                pid4 = (iota + sv * 16) * 4
                plsc.store_scatter(w_v, [pid4], wx0 * wy0)
                plsc.store_scatter(w_v, [pid4 + 1], wx1 * wy0)
                plsc.store_scatter(w_v, [pid4 + 2], wx0 * wy1)
                plsc.store_scatter(w_v, [pid4 + 3], wx1 * wy1)
            for k in range(4):
                pltpu.async_copy(
                    table.at[idx_v.at[k]],
                    rows_v.at[pl.ds(k * GP, GP)],
                    sem,
                )

        def drain_interp(g, b, n=n):
            idx_v, w_v, rows_v, out_v, sem = bufs[b]
            for k in range(4):
                pltpu.make_async_copy(
                    table.at[idx_v.at[k]],
                    rows_v.at[pl.ds(k * GP, GP)],
                    sem,
                ).wait()

            @plsc.parallel_loop(0, GP, step=4, unroll=2)
            def pbody(p):
                wv = w_v[pl.ds(p * 4, 16)]
                for q in range(4):
                    pq = p + q
                    w00 = wv[4 * q]
                    w01 = wv[4 * q + 1]
                    w10 = wv[4 * q + 2]
                    w11 = wv[4 * q + 3]
                    for j in range(C // 16):
                        slc = pl.ds(j * 16, 16)
                        a00 = rows_v[pq, slc]
                        a01 = rows_v[pq + GP, slc]
                        a10 = rows_v[pq + 2 * GP, slc]
                        a11 = rows_v[pq + 3 * GP, slc]
                        out_v[pq, slc] = (
                            a00 * w00 + a01 * w01 + a10 * w10 + a11 * w11
                        )

            p_sample = (wid * NG + g) * GP
            pltpu.sync_copy(out_v, out.at[pl.ds(n * HW + p_sample, GP)])

        idx_fire(0, 0)

        def body(g2, carry):
            gg = 2 * g2
            idx_fire(gg + 1, 1)
            drain_interp(gg, 0)
            idx_fire(gg + 2, 0)
            drain_interp(gg + 1, 1)
            return carry

        lax.fori_loop(0, NG // 2 - 1, body, 0)
        gg = NG - 2
        idx_fire(gg + 1, 1)
        drain_interp(gg, 0)
        drain_interp(gg + 1, 1)
        return ncarry

    lax.fori_loop(0, N, nbody, 0)


@functools.partial(
    pl.kernel,
    out_type=jax.ShapeDtypeStruct((N * HW, CP), jnp.float32),
    mesh=plsc.VectorSubcoreMesh(core_axis_name="c", subcore_axis_name="s"),
    compiler_params=pltpu.CompilerParams(
        needs_layout_passes=False, use_tc_tiling_on_sc=True
    ),
    scratch_types=[
        pltpu.VMEM((NG, 128), jnp.float32),
        pltpu.VMEM((4, GP), jnp.int32),
        pltpu.VMEM((4, GP), jnp.int32),
        pltpu.VMEM((4 * GP,), jnp.float32),
        pltpu.VMEM((4 * GP,), jnp.float32),
        pltpu.VMEM((4 * GP, CP), jnp.float32),
        pltpu.VMEM((4 * GP, CP), jnp.float32),
        pltpu.VMEM((GP, CP), jnp.float32),
        pltpu.VMEM((GP, CP), jnp.float32),
        pltpu.SemaphoreType.DMA,
        pltpu.SemaphoreType.DMA,
    ],
)
def _sc_sample(table, grid_f, out, grid_v, idx_v0, idx_v1, w_v0, w_v1,
               rows_v0, rows_v1, out_v0, out_v1, sem0, sem1):
    _sc_body(table, grid_f, out, grid_v, idx_v0, idx_v1, w_v0, w_v1,
             rows_v0, rows_v1, out_v0, out_v1, sem0, sem1)


def kernel(img, grid):
    table = _make_table(img)
    grid_f = grid.reshape(N * HW * 2 // 128, 128)
    outp = _sc_sample(table, grid_f)
    return _out_transpose(outp)


# speedup vs baseline: 8.3397x; 1.0579x over previous
import functools

import jax
import jax.numpy as jnp
from jax import lax
from jax.experimental import pallas as pl
from jax.experimental.pallas import tpu as pltpu
from jax.experimental.pallas import tpu_sc as plsc

N, C, H, W = 4, 96, 384, 384
CP = 128
HW = H * W
NW = 32
GP = 64
NG = (HW // GP) // NW
HB = 8
TB = HB * W


def _fwd_body(x_ref, o_ref):
    for r in range(HB):
        o_ref[pl.ds(r * W, W), 0:C] = x_ref[0, :, r, :].T


def _make_table(img, n):
    return pl.pallas_call(
        _fwd_body,
        grid=(H // HB,),
        in_specs=[pl.BlockSpec((1, C, HB, W), lambda i, n=n: (n, 0, i, 0))],
        out_specs=pl.BlockSpec((TB, CP), lambda i: (i, 0)),
        out_shape=jax.ShapeDtypeStruct((HW, CP), jnp.float32),
    )(img)


def _bwd_body(x_ref, o_ref):
    for r in range(HB):
        o_ref[0, :, r, :] = x_ref[pl.ds(r * W, W), 0:C].T


def _out_transpose_first(outp):
    return pl.pallas_call(
        _bwd_body,
        grid=(H // HB,),
        in_specs=[pl.BlockSpec((TB, CP), lambda i: (i, 0))],
        out_specs=pl.BlockSpec((1, C, HB, W), lambda i: (0, 0, i, 0)),
        out_shape=jax.ShapeDtypeStruct((N, C, H, W), jnp.float32),
    )(outp)


def _bwd_body_chain(o_any, x_ref, o_ref):
    _bwd_body(x_ref, o_ref)


def _out_transpose_chain(o, outp, n):
    return pl.pallas_call(
        _bwd_body_chain,
        grid=(H // HB,),
        in_specs=[
            pl.BlockSpec(memory_space=pl.ANY),
            pl.BlockSpec((TB, CP), lambda i: (i, 0)),
        ],
        out_specs=pl.BlockSpec((1, C, HB, W), lambda i, n=n: (n, 0, i, 0)),
        out_shape=jax.ShapeDtypeStruct((N, C, H, W), jnp.float32),
        input_output_aliases={0: 0},
    )(o, outp)


def _sc_body(table, grid_f, out, grid_v, idx_v0, idx_v1, w_v0, w_v1,
             rows_v0, rows_v1, out_v0, out_v1, sem0, sem1):
    wid = lax.axis_index("s") * 2 + lax.axis_index("c")
    iota = lax.iota(jnp.int32, 16)
    bufs = ((idx_v0, w_v0, rows_v0, out_v0, sem0),
            (idx_v1, w_v1, rows_v1, out_v1, sem1))

    if True:
        grow = wid * NG
        pltpu.sync_copy(grid_f.at[pl.ds(grow, NG)], grid_v)

        def idx_fire(g, b):
            idx_v, w_v, rows_v, out_v, sem = bufs[b]
            rowv = jnp.zeros((16,), jnp.int32) + g
            for sv in range(GP // 16):
                colv = sv * 32 + 2 * iota
                gxr = plsc.load_gather(grid_v, [rowv, colv])
                gyr = plsc.load_gather(grid_v, [rowv, colv + 1])
                gx = (gxr + 1.0) * (0.5 * (W - 1))
                gy = (gyr + 1.0) * (0.5 * (H - 1))
                x0 = jnp.minimum(gx.astype(jnp.int32), W - 2)
                y0 = jnp.minimum(gy.astype(jnp.int32), H - 2)
                wx1 = gx - x0.astype(jnp.float32)
                wy1 = gy - y0.astype(jnp.float32)
                wx0 = 1.0 - wx1
                wy0 = 1.0 - wy1
                r00 = y0 * W + x0
                sl = pl.ds(sv * 16, 16)
                idx_v[0, sl] = r00
                idx_v[1, sl] = r00 + 1
                idx_v[2, sl] = r00 + W
                idx_v[3, sl] = r00 + (W + 1)
                pid4 = (iota + sv * 16) * 4
                plsc.store_scatter(w_v, [pid4], wx0 * wy0)
                plsc.store_scatter(w_v, [pid4 + 1], wx1 * wy0)
                plsc.store_scatter(w_v, [pid4 + 2], wx0 * wy1)
                plsc.store_scatter(w_v, [pid4 + 3], wx1 * wy1)
            for k in range(4):
                pltpu.async_copy(
                    table.at[idx_v.at[k]],
                    rows_v.at[pl.ds(k * GP, GP)],
                    sem,
                )

        def drain_interp(g, b):
            idx_v, w_v, rows_v, out_v, sem = bufs[b]
            for k in range(4):
                pltpu.make_async_copy(
                    table.at[idx_v.at[k]],
                    rows_v.at[pl.ds(k * GP, GP)],
                    sem,
                ).wait()

            @plsc.parallel_loop(0, GP, step=4, unroll=2)
            def pbody(p):
                wv = w_v[pl.ds(p * 4, 16)]
                for q in range(4):
                    pq = p + q
                    w00 = wv[4 * q]
                    w01 = wv[4 * q + 1]
                    w10 = wv[4 * q + 2]
                    w11 = wv[4 * q + 3]
                    for j in range(C // 16):
                        slc = pl.ds(j * 16, 16)
                        a00 = rows_v[pq, slc]
                        a01 = rows_v[pq + GP, slc]
                        a10 = rows_v[pq + 2 * GP, slc]
                        a11 = rows_v[pq + 3 * GP, slc]
                        out_v[pq, slc] = (
                            a00 * w00 + a01 * w01 + a10 * w10 + a11 * w11
                        )

            p_sample = (wid * NG + g) * GP
            pltpu.sync_copy(out_v, out.at[pl.ds(p_sample, GP)])

        idx_fire(0, 0)

        def body(g2, carry):
            gg = 2 * g2
            idx_fire(gg + 1, 1)
            drain_interp(gg, 0)
            idx_fire(gg + 2, 0)
            drain_interp(gg + 1, 1)
            return carry

        lax.fori_loop(0, NG // 2 - 1, body, 0)
        gg = NG - 2
        idx_fire(gg + 1, 1)
        drain_interp(gg, 0)
        drain_interp(gg + 1, 1)


@functools.partial(
    pl.kernel,
    out_type=jax.ShapeDtypeStruct((HW, CP), jnp.float32),
    mesh=plsc.VectorSubcoreMesh(core_axis_name="c", subcore_axis_name="s"),
    compiler_params=pltpu.CompilerParams(
        needs_layout_passes=False, use_tc_tiling_on_sc=True
    ),
    scratch_types=[
        pltpu.VMEM((NG, 128), jnp.float32),
        pltpu.VMEM((4, GP), jnp.int32),
        pltpu.VMEM((4, GP), jnp.int32),
        pltpu.VMEM((4 * GP,), jnp.float32),
        pltpu.VMEM((4 * GP,), jnp.float32),
        pltpu.VMEM((4 * GP, CP), jnp.float32),
        pltpu.VMEM((4 * GP, CP), jnp.float32),
        pltpu.VMEM((GP, CP), jnp.float32),
        pltpu.VMEM((GP, CP), jnp.float32),
        pltpu.SemaphoreType.DMA,
        pltpu.SemaphoreType.DMA,
    ],
)
def _sc_sample(table, grid_f, out, grid_v, idx_v0, idx_v1, w_v0, w_v1,
               rows_v0, rows_v1, out_v0, out_v1, sem0, sem1):
    _sc_body(table, grid_f, out, grid_v, idx_v0, idx_v1, w_v0, w_v1,
             rows_v0, rows_v1, out_v0, out_v1, sem0, sem1)


def kernel(img, grid):
    grid_f = grid.reshape(N * HW * 2 // 128, 128)
    gpn = HW * 2 // 128
    o = None
    for n in range(N):
        table = _make_table(img, n)
        outp = _sc_sample(table, grid_f[n * gpn:(n + 1) * gpn])
        o = _out_transpose_first(outp) if n == 0 else _out_transpose_chain(o, outp, n)
    return o


# speedup vs baseline: 9.5714x; 1.1477x over previous
import functools

import jax
import jax.numpy as jnp
from jax import lax
from jax.experimental import pallas as pl
from jax.experimental.pallas import tpu as pltpu
from jax.experimental.pallas import tpu_sc as plsc

N, C, H, W = 4, 96, 384, 384
CP = 128
HW = H * W
NW = 32
GP = 64
NG = (HW // GP) // NW
HB = 8
TB = HB * W


def _fwd_body(x_ref, o_ref):
    for r in range(HB):
        o_ref[pl.ds(r * W, W), 0:C] = x_ref[0, :, r, :].T


def _make_table(img, n):
    return pl.pallas_call(
        _fwd_body,
        grid=(H // HB,),
        in_specs=[pl.BlockSpec((1, C, HB, W), lambda i, n=n: (n, 0, i, 0))],
        out_specs=pl.BlockSpec((TB, CP), lambda i: (i, 0)),
        out_shape=jax.ShapeDtypeStruct((HW, CP), jnp.float32),
    )(img)


def _bwd_body(x_ref, o_ref):
    for r in range(HB):
        o_ref[0, :, r, :] = x_ref[pl.ds(r * W, W), 0:C].T


def _out_transpose_first(outp):
    return pl.pallas_call(
        _bwd_body,
        grid=(H // HB,),
        in_specs=[pl.BlockSpec((TB, CP), lambda i: (i, 0))],
        out_specs=pl.BlockSpec((1, C, HB, W), lambda i: (0, 0, i, 0)),
        out_shape=jax.ShapeDtypeStruct((N, C, H, W), jnp.float32),
    )(outp)


def _bwd_body_chain(o_any, x_ref, o_ref):
    _bwd_body(x_ref, o_ref)


def _out_transpose_chain(o, outp, n):
    return pl.pallas_call(
        _bwd_body_chain,
        grid=(H // HB,),
        in_specs=[
            pl.BlockSpec(memory_space=pl.ANY),
            pl.BlockSpec((TB, CP), lambda i: (i, 0)),
        ],
        out_specs=pl.BlockSpec((1, C, HB, W), lambda i, n=n: (n, 0, i, 0)),
        out_shape=jax.ShapeDtypeStruct((N, C, H, W), jnp.float32),
        input_output_aliases={0: 0},
    )(o, outp)


def _sc_body(table, grid_f, out, grid_v, idx_v0, idx_v1, w_v0, w_v1,
             rows_v0, rows_v1, out_v0, out_v1, sem0, sem1):
    wid = lax.axis_index("s") * 2 + lax.axis_index("c")
    iota = lax.iota(jnp.int32, 16)
    bufs = ((idx_v0, w_v0, rows_v0, out_v0, sem0),
            (idx_v1, w_v1, rows_v1, out_v1, sem1))

    if True:
        grow = wid * NG
        pltpu.sync_copy(grid_f.at[pl.ds(grow, NG)], grid_v)

        def idx_fire(g, b):
            idx_v, w_v, rows_v, out_v, sem = bufs[b]
            rowv = jnp.zeros((16,), jnp.int32) + g
            for sv in range(GP // 16):
                colv = sv * 32 + 2 * iota
                gxr = plsc.load_gather(grid_v, [rowv, colv])
                gyr = plsc.load_gather(grid_v, [rowv, colv + 1])
                gx = (gxr + 1.0) * (0.5 * (W - 1))
                gy = (gyr + 1.0) * (0.5 * (H - 1))
                x0 = jnp.minimum(gx.astype(jnp.int32), W - 2)
                y0 = jnp.minimum(gy.astype(jnp.int32), H - 2)
                wx1 = gx - x0.astype(jnp.float32)
                wy1 = gy - y0.astype(jnp.float32)
                wx0 = 1.0 - wx1
                wy0 = 1.0 - wy1
                r00 = y0 * W + x0
                sl = pl.ds(sv * 16, 16)
                idx_v[0, sl] = r00
                idx_v[1, sl] = r00 + 1
                idx_v[2, sl] = r00 + W
                idx_v[3, sl] = r00 + (W + 1)
                pid4 = (iota + sv * 16) * 4
                plsc.store_scatter(w_v, [pid4], wx0 * wy0)
                plsc.store_scatter(w_v, [pid4 + 1], wx1 * wy0)
                plsc.store_scatter(w_v, [pid4 + 2], wx0 * wy1)
                plsc.store_scatter(w_v, [pid4 + 3], wx1 * wy1)
            for k in range(4):
                pltpu.async_copy(
                    table.at[idx_v.at[k]],
                    rows_v.at[pl.ds(k * GP, GP)],
                    sem,
                )

        def drain_interp(g, b):
            idx_v, w_v, rows_v, out_v, sem = bufs[b]
            for k in range(4):
                pltpu.make_async_copy(
                    table.at[idx_v.at[k]],
                    rows_v.at[pl.ds(k * GP, GP)],
                    sem,
                ).wait()

            @plsc.parallel_loop(0, GP, step=4, unroll=2)
            def pbody(p):
                wv = w_v[pl.ds(p * 4, 16)]
                for q in range(4):
                    pq = p + q
                    w00 = wv[4 * q]
                    w01 = wv[4 * q + 1]
                    w10 = wv[4 * q + 2]
                    w11 = wv[4 * q + 3]
                    for j in range(C // 16):
                        slc = pl.ds(j * 16, 16)
                        a00 = rows_v[pq, slc]
                        a01 = rows_v[pq + GP, slc]
                        a10 = rows_v[pq + 2 * GP, slc]
                        a11 = rows_v[pq + 3 * GP, slc]
                        out_v[pq, slc] = (
                            a00 * w00 + a01 * w01 + a10 * w10 + a11 * w11
                        )

            p_sample = (wid * NG + g) * GP
            pltpu.sync_copy(out_v, out.at[pl.ds(p_sample, GP)])

        idx_fire(0, 0)

        def body(g2, carry):
            gg = 2 * g2
            idx_fire(gg + 1, 1)
            drain_interp(gg, 0)
            idx_fire(gg + 2, 0)
            drain_interp(gg + 1, 1)
            return carry

        lax.fori_loop(0, NG // 2 - 1, body, 0)
        gg = NG - 2
        idx_fire(gg + 1, 1)
        drain_interp(gg, 0)
        drain_interp(gg + 1, 1)


@functools.partial(
    pl.kernel,
    out_type=jax.ShapeDtypeStruct((HW, CP), jnp.float32),
    mesh=plsc.VectorSubcoreMesh(core_axis_name="c", subcore_axis_name="s"),
    compiler_params=pltpu.CompilerParams(
        needs_layout_passes=False, use_tc_tiling_on_sc=True
    ),
    scratch_types=[
        pltpu.VMEM((NG, 128), jnp.float32),
        pltpu.VMEM((4, GP), jnp.int32),
        pltpu.VMEM((4, GP), jnp.int32),
        pltpu.VMEM((4 * GP,), jnp.float32),
        pltpu.VMEM((4 * GP,), jnp.float32),
        pltpu.VMEM((4 * GP, CP), jnp.float32),
        pltpu.VMEM((4 * GP, CP), jnp.float32),
        pltpu.VMEM((GP, CP), jnp.float32),
        pltpu.VMEM((GP, CP), jnp.float32),
        pltpu.SemaphoreType.DMA,
        pltpu.SemaphoreType.DMA,
    ],
)
def _sc_sample(table, grid_f, out, grid_v, idx_v0, idx_v1, w_v0, w_v1,
               rows_v0, rows_v1, out_v0, out_v1, sem0, sem1):
    _sc_body(table, grid_f, out, grid_v, idx_v0, idx_v1, w_v0, w_v1,
             rows_v0, rows_v1, out_v0, out_v1, sem0, sem1)


def kernel(img, grid):
    o = None
    for n in range(N):
        table = _make_table(img, n)
        grid_n = grid[n].reshape(HW * 2 // 128, 128)
        outp = _sc_sample(table, grid_n)
        o = _out_transpose_first(outp) if n == 0 else _out_transpose_chain(o, outp, n)
    return o
